# baseline (device time: 14418 ns/iter reference)
import jax
import jax.numpy as jnp
from jax import lax
from jax.experimental import pallas as pl
from jax.experimental.pallas import tpu as pltpu

N_DEV = 8
MASK_ORDER = (1, 3, 4, 2, 5, 7, 6)
N_PEER = len(MASK_ORDER)
C = 8
B = 4


def kernel(x):
    m_per, n = x.shape
    assert m_per % C == 0
    rows = m_per // C

    def body(
        x_ref,
        out_ref,
        buf0,
        buf1,
        buf2,
        buf3,
        acc_ref,
        recv_ref,
        local_sems,
        send_sems,
        recv_sems,
    ):
        my_pos = lax.axis_index("i")
        bufs = (buf0, buf1, buf2, buf3)

        barrier_sem = pltpu.get_barrier_semaphore()
        for m in MASK_ORDER:
            pl.semaphore_signal(
                barrier_sem,
                inc=1,
                device_id=(my_pos ^ m,),
                device_id_type=pl.DeviceIdType.MESH,
            )

        def copy(c):
            return pltpu.make_async_copy(
                x_ref.at[pl.ds(c * rows, rows), :],
                bufs[c % B],
                local_sems.at[c % B],
            )

        for c in range(B):
            copy(c).start()
        acc = jnp.zeros((1, n), jnp.float32)
        for c in range(C):
            copy(c).wait()
            acc = acc + jnp.sum(bufs[c % B][:, :], axis=0, keepdims=True)
            if c + B < C:
                copy(c + B).start()
        acc_ref[:, :] = acc

        pl.semaphore_wait(barrier_sem, N_PEER)
        rdmas = []
        for idx, m in enumerate(MASK_ORDER):
            rdma = pltpu.make_async_remote_copy(
                src_ref=acc_ref,
                dst_ref=recv_ref.at[idx],
                send_sem=send_sems.at[idx],
                recv_sem=recv_sems.at[idx],
                device_id=(my_pos ^ m,),
                device_id_type=pl.DeviceIdType.MESH,
            )
            rdma.start()
            rdmas.append(rdma)
        total = acc_ref[:, :]
        for idx, rdma in enumerate(rdmas):
            rdma.wait_recv()
            total = total + recv_ref[idx, :, :]
        out_ref[:, :] = total
        for rdma in rdmas:
            rdma.wait_send()

    return pl.pallas_call(
        body,
        out_shape=jax.ShapeDtypeStruct((1, n), jnp.float32),
        in_specs=[pl.BlockSpec(memory_space=pl.ANY)],
        out_specs=pl.BlockSpec(memory_space=pltpu.VMEM),
        scratch_shapes=[
            pltpu.VMEM((rows, n), jnp.float32),
            pltpu.VMEM((rows, n), jnp.float32),
            pltpu.VMEM((rows, n), jnp.float32),
            pltpu.VMEM((rows, n), jnp.float32),
            pltpu.VMEM((1, n), jnp.float32),
            pltpu.VMEM((N_PEER, 1, n), jnp.float32),
            pltpu.SemaphoreType.DMA((B,)),
            pltpu.SemaphoreType.DMA((N_PEER,)),
            pltpu.SemaphoreType.DMA((N_PEER,)),
        ],
        compiler_params=pltpu.CompilerParams(collective_id=0),
    )(x)


# device time: 13244 ns/iter; 1.0886x vs baseline; 1.0886x over previous
import jax
import jax.numpy as jnp
from jax import lax
from jax.experimental import pallas as pl
from jax.experimental.pallas import tpu as pltpu

N_DEV = 8
MASK_ORDER = (1, 3, 4, 2, 5, 7, 6)
GRID = 2


def kernel(x):
    m_per, n = x.shape
    assert m_per % GRID == 0
    m_blk = m_per // GRID

    def body(x_ref, out_ref, acc_ref, recv_ref, send_sems, recv_sems):
        g = pl.program_id(0)
        my_pos = lax.axis_index("i")

        @pl.when(g == 0)
        def _():
            barrier_sem = pltpu.get_barrier_semaphore()
            for m in MASK_ORDER:
                pl.semaphore_signal(
                    barrier_sem,
                    inc=1,
                    device_id=(my_pos ^ m,),
                    device_id_type=pl.DeviceIdType.MESH,
                )
            acc_ref[:, :] = jnp.zeros((1, n), jnp.float32)

        acc_ref[:, :] = acc_ref[:, :] + jnp.sum(
            x_ref[:, :], axis=0, keepdims=True
        )

        @pl.when(g == GRID - 1)
        def _():
            pl.semaphore_wait(pltpu.get_barrier_semaphore(), len(MASK_ORDER))
            rdmas = []
            for idx, m in enumerate(MASK_ORDER):
                rdma = pltpu.make_async_remote_copy(
                    src_ref=acc_ref,
                    dst_ref=recv_ref.at[idx],
                    send_sem=send_sems.at[idx],
                    recv_sem=recv_sems.at[idx],
                    device_id=(my_pos ^ m,),
                    device_id_type=pl.DeviceIdType.MESH,
                )
                rdma.start()
                rdmas.append(rdma)
            acc = acc_ref[:, :]
            for idx, rdma in enumerate(rdmas):
                rdma.wait_recv()
                acc = acc + recv_ref[idx, :, :]
            out_ref[:, :] = acc
            for rdma in rdmas:
                rdma.wait_send()

    return pl.pallas_call(
        body,
        grid=(GRID,),
        out_shape=jax.ShapeDtypeStruct((1, n), jnp.float32),
        in_specs=[
            pl.BlockSpec((m_blk, n), lambda g: (g, 0), memory_space=pltpu.VMEM)
        ],
        out_specs=pl.BlockSpec((1, n), lambda g: (0, 0), memory_space=pltpu.VMEM),
        scratch_shapes=[
            pltpu.VMEM((1, n), jnp.float32),
            pltpu.VMEM((len(MASK_ORDER), 1, n), jnp.float32),
            pltpu.SemaphoreType.DMA((len(MASK_ORDER),)),
            pltpu.SemaphoreType.DMA((len(MASK_ORDER),)),
        ],
        compiler_params=pltpu.CompilerParams(collective_id=0),
    )(x)
